# Initial kernel scaffold; baseline (speedup 1.0000x reference)
import jax
import jax.numpy as jnp
from jax.experimental import pallas as pl


def kernel(results, indexes, features, labels):
    raise NotImplementedError("write your pallas kernel here")


# speedup vs baseline: 6.8796x; 6.8796x over previous
import functools

import jax
import jax.numpy as jnp
from jax import lax
from jax.experimental import pallas as pl
from jax.experimental.pallas import tpu as pltpu
from jax.experimental.pallas import tpu_sc as plsc

_M = 100000
_D = 64
_C = 4096
_B = 1024
_TEMP = 0.05

_NW = 32
_CHUNK = 128
_CPW = 25
_M_PAD = _NW * _CPW * _CHUNK
_C_ACC = 4224
_ZROWS = _C_ACC // 16
_CW = 16


def _sc_segment_sum(feat_pad, lbl_pad, indexes, zeros_d, zeros_c, ones_c):
  mesh = plsc.VectorSubcoreMesh(core_axis_name="c", subcore_axis_name="s")

  @functools.partial(
      pl.kernel,
      out_type=[
          jax.ShapeDtypeStruct((2, _C_ACC, _D), jnp.float32),
          jax.ShapeDtypeStruct((2, _C_ACC, _CW), jnp.float32),
          jax.ShapeDtypeStruct((_B,), jnp.int32),
      ],
      mesh=mesh,
      scratch_types=[
          pltpu.VMEM((_CHUNK,), jnp.int32),
          pltpu.VMEM((_CHUNK, _D), jnp.float32),
          pltpu.VMEM((_CHUNK, _CW), jnp.float32),
          pltpu.VMEM((_CHUNK,), jnp.int32),
          pltpu.VMEM((_CHUNK,), jnp.int32),
          pltpu.VMEM_SHARED((_C_ACC, _D), jnp.float32),
          pltpu.VMEM_SHARED((_C_ACC, _CW), jnp.float32),
      ],
  )
  def k(feat_hbm, lbl_hbm, idx_hbm, zd_hbm, zc_hbm, ones_hbm,
        sums_out, cnts_out, tgt_out,
        lbl_v, feat_v, ones_v, idx_v, tgt_v, acc_s, cnt_s):
    cid = lax.axis_index("c")
    sid = lax.axis_index("s")
    wid = sid * 2 + cid

    pltpu.sync_copy(zd_hbm, acc_s.at[pl.ds(sid * _ZROWS, _ZROWS)])
    pltpu.sync_copy(zc_hbm, cnt_s.at[pl.ds(sid * _ZROWS, _ZROWS)])
    pltpu.sync_copy(ones_hbm, ones_v)
    plsc.subcore_barrier()

    def body(j, carry):
      off = (wid * _CPW + j) * _CHUNK
      pltpu.sync_copy(lbl_hbm.at[pl.ds(off, _CHUNK)], lbl_v)
      pltpu.sync_copy(feat_hbm.at[pl.ds(off, _CHUNK)], feat_v)
      pltpu.sync_copy(feat_v, acc_s.at[lbl_v], add=True)
      pltpu.sync_copy(ones_v, cnt_s.at[lbl_v], add=True)
      return carry

    lax.fori_loop(0, _CPW, body, 0)
    plsc.subcore_barrier()

    row = pl.ds(sid * _ZROWS, _ZROWS)
    pltpu.sync_copy(acc_s.at[row], sums_out.at[cid].at[row])
    pltpu.sync_copy(cnt_s.at[row], cnts_out.at[cid].at[row])

    @pl.when(wid < _B // _CHUNK)
    def _():
      boff = wid * _CHUNK
      pltpu.sync_copy(idx_hbm.at[pl.ds(boff, _CHUNK)], idx_v)
      pltpu.sync_copy(lbl_hbm.at[idx_v], tgt_v)
      pltpu.sync_copy(tgt_v, tgt_out.at[pl.ds(boff, _CHUNK)])

  return k(feat_pad, lbl_pad, indexes, zeros_d, zeros_c, ones_c)


_CBLK = 512


def _tc_body(x_ref, s_ref, c_ref, t_ref, o_ref, rs_acc, tv_acc):
  i = pl.program_id(0)
  x = x_ref[...]
  nrm = jnp.sqrt(jnp.sum(x * x, axis=1, keepdims=True))
  xn = x / jnp.maximum(nrm, 1e-12)
  s = s_ref[...]
  f = s[0] + s[1]
  c = c_ref[...]
  cnt = c[0, :, 0] + c[1, :, 0]
  logits = lax.dot_general(xn, f, (((1,), (1,)), ((), ())),
                           preferred_element_type=jnp.float32)
  denom = _TEMP * jnp.where(cnt > 0, cnt, 1.0)
  vec = logits / denom[None, :]
  e = jnp.exp(vec) * (cnt > 0).astype(jnp.float32)[None, :]
  colid = i * _CBLK + lax.broadcasted_iota(jnp.int32, (_B, _CBLK), 1)
  tmask = (colid == t_ref[...]).astype(jnp.float32)
  ps = jnp.sum(e, axis=1, keepdims=True)
  pt = jnp.sum(e * tmask, axis=1, keepdims=True)

  @pl.when(i == 0)
  def _():
    rs_acc[...] = ps
    tv_acc[...] = pt

  @pl.when(i > 0)
  def _():
    rs_acc[...] += ps
    tv_acc[...] += pt

  @pl.when(i == pl.num_programs(0) - 1)
  def _():
    tot = rs_acc[...] + 1e-6
    logp = jnp.log(tv_acc[...] / tot + 1e-6)
    o_ref[...] = jnp.mean(-logp).reshape(1, 1)


def _tc_loss(results, sums, cnts, targets):
  return pl.pallas_call(
      _tc_body,
      grid=(_C // _CBLK,),
      in_specs=[
          pl.BlockSpec((_B, _D), lambda i: (0, 0)),
          pl.BlockSpec((2, _CBLK, _D), lambda i: (0, i, 0)),
          pl.BlockSpec((2, _CBLK, _CW), lambda i: (0, i, 0)),
          pl.BlockSpec((_B, 1), lambda i: (0, 0)),
      ],
      out_specs=pl.BlockSpec((1, 1), lambda i: (0, 0)),
      out_shape=jax.ShapeDtypeStruct((1, 1), jnp.float32),
      scratch_shapes=[
          pltpu.VMEM((_B, 1), jnp.float32),
          pltpu.VMEM((_B, 1), jnp.float32),
      ],
  )(results, sums, cnts, targets.reshape(_B, 1))


def kernel(results, indexes, features, labels):
  pad = _M_PAD - _M
  feat_pad = jnp.concatenate(
      [features, jnp.zeros((pad, _D), jnp.float32)], axis=0)
  lbl_pad = jnp.concatenate(
      [labels.astype(jnp.int32), jnp.full((pad,), _C, jnp.int32)], axis=0)
  zeros_d = jnp.zeros((_ZROWS, _D), jnp.float32)
  zeros_c = jnp.zeros((_ZROWS, _CW), jnp.float32)
  ones_c = jnp.ones((_CHUNK, _CW), jnp.float32)
  sums, cnts, targets = _sc_segment_sum(
      feat_pad, lbl_pad, indexes.astype(jnp.int32), zeros_d, zeros_c, ones_c)
  out = _tc_loss(results, sums, cnts, targets)
  return out[0, 0]


# speedup vs baseline: 9.7787x; 1.4214x over previous
import functools

import jax
import jax.numpy as jnp
from jax import lax
from jax.experimental import pallas as pl
from jax.experimental.pallas import tpu as pltpu
from jax.experimental.pallas import tpu_sc as plsc

_M = 100000
_D = 64
_C = 4096
_B = 1024
_TEMP = 0.05

_NW = 32
_CHUNK = 128
_NFULL = _M // _CHUNK
_TAIL = _M - _NFULL * _CHUNK
_JMAX = (_NFULL + _NW - 1) // _NW
_C_ACC = _C
_ZROWS = _C_ACC // 16
_CW = 16


def _sc_segment_sum(features, labels, indexes, zeros_d, zeros_c, ones_c):
  mesh = plsc.VectorSubcoreMesh(core_axis_name="c", subcore_axis_name="s")

  @functools.partial(
      pl.kernel,
      out_type=[
          jax.ShapeDtypeStruct((2, _C_ACC, _D), jnp.float32),
          jax.ShapeDtypeStruct((2, _C_ACC, _CW), jnp.float32),
          jax.ShapeDtypeStruct((_B,), jnp.int32),
      ],
      mesh=mesh,
      scratch_types=[
          pltpu.VMEM((2, _CHUNK), jnp.int32),
          pltpu.VMEM((2, _CHUNK, _D), jnp.float32),
          pltpu.VMEM((_CHUNK, _CW), jnp.float32),
          pltpu.VMEM((1, _TAIL), jnp.int32),
          pltpu.VMEM((_TAIL, _D), jnp.float32),
          pltpu.VMEM((_CHUNK,), jnp.int32),
          pltpu.VMEM((_CHUNK,), jnp.int32),
          pltpu.SemaphoreType.DMA((2,)),
          pltpu.SemaphoreType.DMA((2,)),
          pltpu.VMEM_SHARED((_C_ACC, _D), jnp.float32),
          pltpu.VMEM_SHARED((_C_ACC, _CW), jnp.float32),
      ],
  )
  def k(feat_hbm, lbl_hbm, idx_hbm, zd_hbm, zc_hbm, ones_hbm,
        sums_out, cnts_out, tgt_out,
        lbl_v, feat_v, ones_v, tl_v, tf_v, idx_v, tgt_v,
        lsem, fsem, acc_s, cnt_s):
    cid = lax.axis_index("c")
    sid = lax.axis_index("s")
    wid = sid * 2 + cid

    pltpu.sync_copy(zd_hbm, acc_s.at[pl.ds(sid * _ZROWS, _ZROWS)])
    pltpu.sync_copy(zc_hbm, cnt_s.at[pl.ds(sid * _ZROWS, _ZROWS)])
    pltpu.sync_copy(ones_hbm, ones_v)
    plsc.subcore_barrier()

    def start_load(j):
      c = wid + _NW * j
      b = j % 2
      off = c * _CHUNK
      pltpu.make_async_copy(
          lbl_hbm.at[pl.ds(off, _CHUNK)], lbl_v.at[b], lsem.at[b]).start()
      pltpu.make_async_copy(
          feat_hbm.at[pl.ds(off, _CHUNK)], feat_v.at[b], fsem.at[b]).start()

    def wait_load(j):
      c = wid + _NW * j
      b = j % 2
      off = c * _CHUNK
      pltpu.make_async_copy(
          lbl_hbm.at[pl.ds(off, _CHUNK)], lbl_v.at[b], lsem.at[b]).wait()
      pltpu.make_async_copy(
          feat_hbm.at[pl.ds(off, _CHUNK)], feat_v.at[b], fsem.at[b]).wait()

    @pl.when(wid < _NFULL)
    def _():
      start_load(0)

    def body(j, carry):
      @pl.when(wid + _NW * (j + 1) < _NFULL)
      def _():
        start_load(j + 1)

      @pl.when(wid + _NW * j < _NFULL)
      def _():
        wait_load(j)
        b = j % 2
        pltpu.sync_copy(feat_v.at[b], acc_s.at[lbl_v.at[b]], add=True)
        pltpu.sync_copy(ones_v, cnt_s.at[lbl_v.at[b]], add=True)

      return carry

    lax.fori_loop(0, _JMAX, body, 0)

    @pl.when(wid == _NW - 1)
    def _():
      off = _NFULL * _CHUNK
      pltpu.sync_copy(lbl_hbm.at[pl.ds(off, _TAIL)], tl_v.at[0])
      pltpu.sync_copy(feat_hbm.at[pl.ds(off, _TAIL)], tf_v)
      pltpu.sync_copy(tf_v, acc_s.at[tl_v.at[0]], add=True)
      pltpu.sync_copy(ones_v.at[pl.ds(0, _TAIL)], cnt_s.at[tl_v.at[0]],
                      add=True)

    plsc.subcore_barrier()

    row = pl.ds(sid * _ZROWS, _ZROWS)
    pltpu.sync_copy(acc_s.at[row], sums_out.at[cid].at[row])
    pltpu.sync_copy(cnt_s.at[row], cnts_out.at[cid].at[row])

    @pl.when(wid < _B // _CHUNK)
    def _():
      boff = wid * _CHUNK
      pltpu.sync_copy(idx_hbm.at[pl.ds(boff, _CHUNK)], idx_v)
      pltpu.sync_copy(lbl_hbm.at[idx_v], tgt_v)
      pltpu.sync_copy(tgt_v, tgt_out.at[pl.ds(boff, _CHUNK)])

  return k(features, labels, indexes, zeros_d, zeros_c, ones_c)


_CBLK = 512


def _tc_body(x_ref, s_ref, c_ref, t_ref, o_ref, rs_acc, tv_acc):
  i = pl.program_id(0)
  x = x_ref[...]
  nrm = jnp.sqrt(jnp.sum(x * x, axis=1, keepdims=True))
  xn = x / jnp.maximum(nrm, 1e-12)
  s = s_ref[...]
  f = s[0] + s[1]
  c = c_ref[...]
  cnt = c[0, :, 0] + c[1, :, 0]
  logits = lax.dot_general(xn, f, (((1,), (1,)), ((), ())),
                           preferred_element_type=jnp.float32)
  denom = _TEMP * jnp.where(cnt > 0, cnt, 1.0)
  vec = logits / denom[None, :]
  e = jnp.exp(vec) * (cnt > 0).astype(jnp.float32)[None, :]
  colid = i * _CBLK + lax.broadcasted_iota(jnp.int32, (_B, _CBLK), 1)
  tmask = (colid == t_ref[...]).astype(jnp.float32)
  ps = jnp.sum(e, axis=1, keepdims=True)
  pt = jnp.sum(e * tmask, axis=1, keepdims=True)

  @pl.when(i == 0)
  def _():
    rs_acc[...] = ps
    tv_acc[...] = pt

  @pl.when(i > 0)
  def _():
    rs_acc[...] += ps
    tv_acc[...] += pt

  @pl.when(i == pl.num_programs(0) - 1)
  def _():
    tot = rs_acc[...] + 1e-6
    logp = jnp.log(tv_acc[...] / tot + 1e-6)
    o_ref[...] = jnp.mean(-logp).reshape(1, 1)


def _tc_loss(results, sums, cnts, targets):
  return pl.pallas_call(
      _tc_body,
      grid=(_C // _CBLK,),
      in_specs=[
          pl.BlockSpec((_B, _D), lambda i: (0, 0)),
          pl.BlockSpec((2, _CBLK, _D), lambda i: (0, i, 0)),
          pl.BlockSpec((2, _CBLK, _CW), lambda i: (0, i, 0)),
          pl.BlockSpec((_B, 1), lambda i: (0, 0)),
      ],
      out_specs=pl.BlockSpec((1, 1), lambda i: (0, 0)),
      out_shape=jax.ShapeDtypeStruct((1, 1), jnp.float32),
      scratch_shapes=[
          pltpu.VMEM((_B, 1), jnp.float32),
          pltpu.VMEM((_B, 1), jnp.float32),
      ],
  )(results, sums, cnts, targets.reshape(_B, 1))


def kernel(results, indexes, features, labels):
  zeros_d = jnp.zeros((_ZROWS, _D), jnp.float32)
  zeros_c = jnp.zeros((_ZROWS, _CW), jnp.float32)
  ones_c = jnp.ones((_CHUNK, _CW), jnp.float32)
  sums, cnts, targets = _sc_segment_sum(
      features, labels.astype(jnp.int32), indexes.astype(jnp.int32),
      zeros_d, zeros_c, ones_c)
  out = _tc_loss(results, sums, cnts, targets)
  return out[0, 0]
